# Initial kernel scaffold; baseline (speedup 1.0000x reference)
import jax
import jax.numpy as jnp
from jax.experimental import pallas as pl


def kernel(x, edge_index, node_kind, family_ids, kind_emb, W1s, W1n, b1, W2s, W2n, b2, Wp1, bp1, Wp2, bp2):
    raise NotImplementedError("write your pallas kernel here")


# speedup vs baseline: 3.4803x; 3.4803x over previous
import functools

import jax
import jax.numpy as jnp
from jax import lax
from jax.experimental import pallas as pl
from jax.experimental.pallas import tpu as pltpu
from jax.experimental.pallas import tpu_sc as plsc

N = 10000
E = 320000
D_IN = 128
D_H = 256
D_OUT = 128
D_PRED = 512
N_KINDS = 4

NC = 2
NS = 16
NW = NC * NS
L = 16

CH = 128
NCHUNK = -(-E // (NW * CH))
EPAD = NW * NCHUNK * CH
NPAD = 10240
NROW = NPAD // CH
RPT = NPAD // NS
OH = 64


def _sc_agg_body(with_deg, *refs):
    if with_deg:
        (table, src3, dst3, zeros, eye, out, degout,
         src_v, dst_v, rows_v, oh_v, dr_v, dc_v, sem, sem2, acc_sh, deg_sh) = refs
    else:
        (table, src3, dst3, zeros, out,
         src_v, dst_v, rows_v, sem, acc_sh) = refs

    c = lax.axis_index("c")
    s = lax.axis_index("s")
    w = c * NS + s

    pltpu.sync_copy(zeros.at[pl.ds(s * RPT, RPT)], acc_sh.at[pl.ds(s * RPT, RPT)])
    pltpu.sync_copy(src3.at[w], src_v)
    pltpu.sync_copy(dst3.at[w], dst_v)
    if with_deg:
        @pl.when(s == 0)
        def _():
            pltpu.sync_copy(zeros.at[pl.ds(0, NROW)], deg_sh)
    plsc.subcore_barrier()

    def chunk(j, carry):
        cp1 = pltpu.async_copy(table.at[src_v.at[j]], rows_v, sem)
        if with_deg:
            for h in range(2):
                for t in range(OH // L):
                    d16 = dst_v[j, pl.ds(h * OH + t * L, L)]
                    dr_v[h, pl.ds(t * L, L)] = d16 >> 7
                    dc_v[h, pl.ds(t * L, L)] = d16 & 127
        cp1.wait()
        pltpu.sync_copy(rows_v, acc_sh.at[dst_v.at[j]], add=True)
        if with_deg:
            for h in range(2):
                pltpu.async_copy(eye.at[dc_v.at[h]], oh_v, sem2).wait()
                pltpu.sync_copy(oh_v, deg_sh.at[dr_v.at[h]], add=True)
        return carry

    lax.fori_loop(0, NCHUNK, chunk, 0)
    plsc.subcore_barrier()

    pltpu.sync_copy(acc_sh.at[pl.ds(s * RPT, RPT)], out.at[c, pl.ds(s * RPT, RPT)])
    if with_deg:
        @pl.when(s == 0)
        def _():
            pltpu.sync_copy(deg_sh, degout.at[c])


def _make_sc_agg(with_deg):
    mesh = plsc.VectorSubcoreMesh(core_axis_name="c", subcore_axis_name="s")
    sums = jax.ShapeDtypeStruct((NC, NPAD, D_IN), jnp.float32)
    outs = [sums]
    scratch = [
        pltpu.VMEM((NCHUNK, CH), jnp.int32),
        pltpu.VMEM((NCHUNK, CH), jnp.int32),
        pltpu.VMEM((CH, D_IN), jnp.float32),
    ]
    if with_deg:
        outs.append(jax.ShapeDtypeStruct((NC, NROW, CH), jnp.float32))
        scratch += [
            pltpu.VMEM((OH, CH), jnp.float32),
            pltpu.VMEM((2, OH), jnp.int32),
            pltpu.VMEM((2, OH), jnp.int32),
        ]
    scratch.append(pltpu.SemaphoreType.DMA)
    if with_deg:
        scratch.append(pltpu.SemaphoreType.DMA)
    scratch.append(pltpu.VMEM_SHARED((NPAD, D_IN), jnp.float32))
    if with_deg:
        scratch.append(pltpu.VMEM_SHARED((NROW, CH), jnp.float32))
    return pl.kernel(
        functools.partial(_sc_agg_body, with_deg),
        out_type=outs if with_deg else sums,
        mesh=mesh,
        scratch_types=scratch,
    )


_sc_agg_deg = _make_sc_agg(True)
_sc_agg = _make_sc_agg(False)


BLK = 400
NBLK = N // BLK

_dot = functools.partial(jnp.dot, preferred_element_type=jnp.float32,
                         precision=lax.Precision.HIGHEST)


def _stage_a_body(x, nk, emb, h0):
    h = x[...]
    k = nk[...].astype(jnp.float32)
    for kk in range(N_KINDS):
        mask = jnp.where(k == kk, 1.0, 0.0)
        h = h + mask * emb[kk, :][None, :]
    h0[...] = h


def _tc_stage_a(x, nk2, emb):
    return pl.pallas_call(
        _stage_a_body,
        grid=(NBLK,),
        in_specs=[
            pl.BlockSpec((BLK, D_IN), lambda i: (i, 0)),
            pl.BlockSpec((BLK, 1), lambda i: (i, 0)),
            pl.BlockSpec((N_KINDS, D_IN), lambda i: (0, 0)),
        ],
        out_specs=pl.BlockSpec((BLK, D_IN), lambda i: (i, 0)),
        out_shape=jax.ShapeDtypeStruct((N, D_IN), jnp.float32),
    )(x, nk2, emb)


DBLK = NPAD // 8


def _deg_body(degp, out):
    out[...] = jnp.maximum(jnp.sum(degp[...], axis=0), 1.0)[:, None]


def _tc_deg(degp):
    return pl.pallas_call(
        _deg_body,
        grid=(8,),
        in_specs=[pl.BlockSpec((NC, DBLK), lambda i: (0, i))],
        out_specs=pl.BlockSpec((DBLK, 1), lambda i: (i, 0)),
        out_shape=jax.ShapeDtypeStruct((NPAD, 1), jnp.float32),
    )(degp)


def _stage_b_body(h0, s0a, s0b, dg, w1s, w1n, b1, w2s, w2n, z, hs):
    deg = dg[...]
    a0 = (s0a[...] + s0b[...]) / deg
    h1 = jax.nn.relu(_dot(h0[...], w1s[...]) + _dot(a0, w1n[...]) + b1[...])
    z[...] = _dot(h1, w2n[...])
    hs[...] = _dot(h1, w2s[...])


def _tc_stage_b(h0, s0a, s0b, dg, W1s, W1n, b1, W2s, W2n):
    return pl.pallas_call(
        _stage_b_body,
        grid=(NBLK,),
        in_specs=[
            pl.BlockSpec((BLK, D_IN), lambda i: (i, 0)),
            pl.BlockSpec((BLK, D_IN), lambda i: (i, 0)),
            pl.BlockSpec((BLK, D_IN), lambda i: (i, 0)),
            pl.BlockSpec((BLK, 1), lambda i: (i, 0)),
            pl.BlockSpec((D_IN, D_H), lambda i: (0, 0)),
            pl.BlockSpec((D_IN, D_H), lambda i: (0, 0)),
            pl.BlockSpec((1, D_H), lambda i: (0, 0)),
            pl.BlockSpec((D_H, D_OUT), lambda i: (0, 0)),
            pl.BlockSpec((D_H, D_OUT), lambda i: (0, 0)),
        ],
        out_specs=[
            pl.BlockSpec((BLK, D_OUT), lambda i: (i, 0)),
            pl.BlockSpec((BLK, D_OUT), lambda i: (i, 0)),
        ],
        out_shape=[
            jax.ShapeDtypeStruct((N, D_OUT), jnp.float32),
            jax.ShapeDtypeStruct((N, D_OUT), jnp.float32),
        ],
    )(h0, s0a, s0b, dg, W1s, W1n, b1, W2s, W2n)


def _stage_c_body(hs, s1a, s1b, dg, b2, wp1, bp1, wp2, bp2, p):
    deg = dg[...]
    a1 = (s1a[...] + s1b[...]) / deg
    h2 = hs[...] + a1 + b2[...]
    g = jax.nn.relu(_dot(h2, wp1[...]) + bp1[...])
    p[...] = _dot(g, wp2[...]) + bp2[...]


def _tc_stage_c(hs, s1a, s1b, dg, b2, Wp1, bp1, Wp2, bp2):
    return pl.pallas_call(
        _stage_c_body,
        grid=(NBLK,),
        in_specs=[
            pl.BlockSpec((BLK, D_OUT), lambda i: (i, 0)),
            pl.BlockSpec((BLK, D_OUT), lambda i: (i, 0)),
            pl.BlockSpec((BLK, D_OUT), lambda i: (i, 0)),
            pl.BlockSpec((BLK, 1), lambda i: (i, 0)),
            pl.BlockSpec((1, D_OUT), lambda i: (0, 0)),
            pl.BlockSpec((D_OUT, D_PRED), lambda i: (0, 0)),
            pl.BlockSpec((1, D_PRED), lambda i: (0, 0)),
            pl.BlockSpec((D_PRED, D_OUT), lambda i: (0, 0)),
            pl.BlockSpec((1, D_OUT), lambda i: (0, 0)),
        ],
        out_specs=pl.BlockSpec((BLK, D_OUT), lambda i: (i, 0)),
        out_shape=jax.ShapeDtypeStruct((N, D_OUT), jnp.float32),
    )(hs, s1a, s1b, dg, b2, Wp1, bp1, Wp2, bp2)


def kernel(x, edge_index, node_kind, family_ids, kind_emb,
           W1s, W1n, b1, W2s, W2n, b2, Wp1, bp1, Wp2, bp2):
    src = edge_index[0]
    dst = edge_index[1]
    pad = EPAD - E
    src3 = jnp.concatenate([src, jnp.zeros((pad,), jnp.int32)]).reshape(NW, NCHUNK, CH)
    dst3 = jnp.concatenate([dst, jnp.full((pad,), N, jnp.int32)]).reshape(NW, NCHUNK, CH)
    zeros = jnp.zeros((NPAD, D_IN), jnp.float32)
    eye = jnp.eye(CH, dtype=jnp.float32)

    nk2 = node_kind[:, None]
    b1r = b1[None, :]
    b2r = b2[None, :]
    bp1r = bp1[None, :]
    bp2r = bp2[None, :]

    h0 = _tc_stage_a(x, nk2, kind_emb)
    s0, deg = _sc_agg_deg(h0, src3, dst3, zeros, eye)
    dg = _tc_deg(deg.reshape(NC, NPAD))
    z, hs = _tc_stage_b(h0, s0[0], s0[1], dg, W1s, W1n, b1r, W2s, W2n)
    s1 = _sc_agg(z, src3, dst3, zeros)
    p = _tc_stage_c(hs, s1[0], s1[1], dg, b2r, Wp1, bp1r, Wp2, bp2r)
    return p
